# baseline (speedup 1.0000x reference)
import functools

import jax
import jax.numpy as jnp
from jax import lax
from jax.experimental import pallas as pl
from jax.experimental.pallas import tpu as pltpu, tpu_sc as plsc

B = 4096
N = 200
M = 26
DIM = 64
NC = 2
NS = 16
NW = NC * NS
RPW = B // NW
HALF = N // 2
HPAD = 104
LANES = 16
NCH = DIM // LANES


NB = 4


def _body(inp_hbm, offs_hbm, psw_hbm, table_hbm, out_hbm,
          inp_v, offs_v, psw_v, rows_v, cs_v, out_v,
          sem0, sem1, sem2, sem3, osem0, osem1):
    wid = lax.axis_index("s") * NC + lax.axis_index("c")
    base = wid * RPW
    sems = (sem0, sem1, sem2, sem3)
    osems = (osem0, osem1)

    pltpu.sync_copy(inp_hbm.at[pl.ds(base, RPW)], inp_v)
    pltpu.sync_copy(offs_hbm.at[pl.ds(base, RPW)], offs_v)
    pltpu.sync_copy(psw_hbm.at[pl.ds(base, RPW)], psw_v)

    zero = jnp.zeros((LANES,), jnp.float32)
    lanes = lax.iota(jnp.int32, LANES)
    _bcast_dn = lax.GatherDimensionNumbers(
        offset_dims=(), collapsed_slice_dims=(0,), start_index_map=(0,)
    )

    def splat(x):
        return jnp.full((LANES,), x, jnp.int32)

    def bcast_lane(v, l):
        idx = jnp.full((LANES, 1), l, jnp.int32)
        return lax.gather(v, idx, _bcast_dn, (1,),
                          mode=lax.GatherScatterMode.PROMISE_IN_BOUNDS)

    for c in range(NCH):
        cs_v[0, pl.ds(LANES * c, LANES)] = zero

    def start_gather(r, buf):
        pltpu.async_copy(table_hbm.at[inp_v.at[r, 0]], rows_v.at[buf, 0], sems[buf])
        pltpu.async_copy(table_hbm.at[inp_v.at[r, 1]], rows_v.at[buf, 1], sems[buf])

    def wait_gather(buf):
        for k in range(2):
            pltpu.make_async_copy(
                table_hbm.at[inp_v.at[0, k]], rows_v.at[buf, k], sems[buf]
            ).wait()

    for b in range(NB - 1):
        start_gather(b, b)

    def g_body(g, _):
        for p in range(NB):
            r = NB * g + p
            phase = p % 2
            wait_gather(p)

            @pl.when(r + NB - 1 < RPW)
            def _():
                start_gather(r + NB - 1, (p + NB - 1) % NB)

            accs = [zero] * NCH
            for k in range(2):
                for chunk in range((HALF + LANES - 1) // LANES):
                    jbase = chunk * LANES
                    cnt = min(LANES, HALF - jbase)
                    wv16 = psw_v[r, pl.ds(k * HALF + jbase, LANES)]
                    for l in range(cnt):
                        j = jbase + l
                        w = bcast_lane(wv16, l)
                        for c in range(NCH):
                            x = rows_v[p, k, j, pl.ds(LANES * c, LANES)]
                            a = accs[c] + x * w
                            cs_v[k * HALF + j + 1, pl.ds(LANES * c, LANES)] = a
                            accs[c] = a

            if p < 2:
                @pl.when(g > 0)
                def _():
                    pltpu.make_async_copy(
                        out_v.at[phase], out_hbm.at[base], osems[phase]
                    ).wait()
            else:
                pltpu.make_async_copy(
                    out_v.at[phase], out_hbm.at[base], osems[phase]
                ).wait()
            prev = [zero] * NCH
            for m in range(M):
                offm = plsc.load_gather(offs_v, [splat(r), splat(m)]) + 1
                for c in range(NCH):
                    cur = plsc.load_gather(cs_v, [offm, lanes + LANES * c])
                    out_v[phase, m, pl.ds(LANES * c, LANES)] = cur - prev[c]
                    prev[c] = cur
            pltpu.async_copy(out_v.at[phase], out_hbm.at[base + r], osems[phase])
        return 0

    lax.fori_loop(0, RPW // NB, g_body, 0)
    for phase in range(2):
        pltpu.make_async_copy(out_v.at[phase], out_hbm.at[base], osems[phase]).wait()


@functools.partial(
    pl.kernel,
    out_type=jax.ShapeDtypeStruct((B, M, DIM), jnp.float32),
    mesh=plsc.VectorSubcoreMesh(
        core_axis_name="c", subcore_axis_name="s", num_cores=NC, num_subcores=NS
    ),
    scratch_types=[
        pltpu.VMEM((RPW, 2, HPAD), jnp.int32),
        pltpu.VMEM((RPW, M), jnp.int32),
        pltpu.VMEM((RPW, N + LANES), jnp.float32),
        pltpu.VMEM((NB, 2, HPAD, DIM), jnp.float32),
        pltpu.VMEM((N + 4, DIM), jnp.float32),
        pltpu.VMEM((2, M, DIM), jnp.float32),
        pltpu.SemaphoreType.DMA,
        pltpu.SemaphoreType.DMA,
        pltpu.SemaphoreType.DMA,
        pltpu.SemaphoreType.DMA,
        pltpu.SemaphoreType.DMA,
        pltpu.SemaphoreType.DMA,
    ],
    compiler_params=pltpu.CompilerParams(
        use_tc_tiling_on_sc=False, needs_layout_passes=False
    ),
)
def _embedding_bag_sc(inp_hbm, offs_hbm, psw_hbm, table_hbm, out_hbm,
                      inp_v, offs_v, psw_v, rows_v, cs_v, out_v,
                      sem0, sem1, sem2, sem3, osem0, osem1):
    _body(inp_hbm, offs_hbm, psw_hbm, table_hbm, out_hbm,
          inp_v, offs_v, psw_v, rows_v, cs_v, out_v,
          sem0, sem1, sem2, sem3, osem0, osem1)


def kernel(input, offsets, per_sample_weights, weight):
    inp_pad = jnp.pad(input.reshape(B, 2, HALF), ((0, 0), (0, 0), (0, HPAD - HALF)))
    psw_pad = jnp.pad(per_sample_weights, ((0, 0), (0, LANES)))
    score = _embedding_bag_sc(inp_pad, offsets, psw_pad, weight)
    return score, jnp.float32(0.0)


# speedup vs baseline: 1.0003x; 1.0003x over previous
import functools

import jax
import jax.numpy as jnp
from jax import lax
from jax.experimental import pallas as pl
from jax.experimental.pallas import tpu as pltpu, tpu_sc as plsc

B = 4096
N = 200
M = 26
DIM = 64
NC = 2
NS = 16
NW = NC * NS
RPW = B // NW
HALF = N // 2
HPAD = 104
LANES = 16
NCH = DIM // LANES
NB = 4

def _body(inp_hbm, offs_hbm, psw_hbm, table_hbm, out_hbm,
          inp_v, offs_v, psw_v, rows_v, cs_v, out_v,
          gsems, osems):
    wid = lax.axis_index("s") * NC + lax.axis_index("c")
    base = wid * RPW

    pltpu.sync_copy(inp_hbm.at[pl.ds(base, RPW)], inp_v)
    pltpu.sync_copy(offs_hbm.at[pl.ds(base, RPW)], offs_v)
    pltpu.sync_copy(psw_hbm.at[pl.ds(base, RPW)], psw_v)

    zero = jnp.zeros((LANES,), jnp.float32)
    lanes = lax.iota(jnp.int32, LANES)
    _bcast_dn = lax.GatherDimensionNumbers(
        offset_dims=(), collapsed_slice_dims=(0,), start_index_map=(0,)
    )

    def splat(x):
        return jnp.full((LANES,), x, jnp.int32)

    def bcast_lane(v, l):
        idx = jnp.full((LANES, 1), l, jnp.int32)
        return lax.gather(v, idx, _bcast_dn, (1,),
                          mode=lax.GatherScatterMode.PROMISE_IN_BOUNDS)

    for c in range(NCH):
        cs_v[0, pl.ds(LANES * c, LANES)] = zero

    def start_gather(r, buf):
        for k in range(2):
            pltpu.async_copy(
                table_hbm.at[inp_v.at[r, k]], rows_v.at[buf, k], gsems[buf]
            )

    def wait_gather(buf):
        for k in range(2):
            pltpu.make_async_copy(
                table_hbm.at[inp_v.at[0, k]], rows_v.at[buf, k], gsems[buf]
            ).wait()

    for b in range(NB - 1):
        start_gather(b, b)

    def g_body(g, _):
        for p in range(NB):
            r = NB * g + p
            phase = p % 2
            wait_gather(p)

            @pl.when(r + NB - 1 < RPW)
            def _():
                start_gather(r + NB - 1, (p + NB - 1) % NB)

            accs = [zero] * NCH
            for k in range(2):
                for chunk in range((HALF + LANES - 1) // LANES):
                    jbase = chunk * LANES
                    cnt = min(LANES, HALF - jbase)
                    wv16 = psw_v[r, pl.ds(k * HALF + jbase, LANES)]
                    for l in range(cnt):
                        jj = jbase + l
                        j = k * HALF + jj
                        w = bcast_lane(wv16, l)
                        for c in range(NCH):
                            x = rows_v[p, k, jj, pl.ds(LANES * c, LANES)]
                            a = accs[c] + x * w
                            cs_v[j + 1, pl.ds(LANES * c, LANES)] = a
                            accs[c] = a

            @pl.when(r >= 2)
            def _():
                pltpu.make_async_copy(
                    out_v.at[phase], out_hbm.at[base], osems[phase]
                ).wait()
            prev = [zero] * NCH
            for m in range(M):
                offm = plsc.load_gather(offs_v, [splat(r), splat(m)]) + 1
                for c in range(NCH):
                    cur = plsc.load_gather(cs_v, [offm, lanes + LANES * c])
                    out_v[phase, m, pl.ds(LANES * c, LANES)] = cur - prev[c]
                    prev[c] = cur
            pltpu.async_copy(out_v.at[phase], out_hbm.at[base + r], osems[phase])
        return 0

    lax.fori_loop(0, RPW // NB, g_body, 0)

    for phase in range(2):
        pltpu.make_async_copy(out_v.at[phase], out_hbm.at[base], osems[phase]).wait()


@functools.partial(
    pl.kernel,
    out_type=jax.ShapeDtypeStruct((B, M, DIM), jnp.float32),
    mesh=plsc.VectorSubcoreMesh(
        core_axis_name="c", subcore_axis_name="s", num_cores=NC, num_subcores=NS
    ),
    scratch_types=[
        pltpu.VMEM((RPW, 2, HPAD), jnp.int32),
        pltpu.VMEM((RPW, M), jnp.int32),
        pltpu.VMEM((RPW, N + LANES), jnp.float32),
        pltpu.VMEM((NB, 2, HPAD, DIM), jnp.float32),
        pltpu.VMEM((N + 4, DIM), jnp.float32),
        pltpu.VMEM((2, M, DIM), jnp.float32),
        [pltpu.SemaphoreType.DMA] * NB,
        [pltpu.SemaphoreType.DMA] * 2,
    ],
    compiler_params=pltpu.CompilerParams(
        use_tc_tiling_on_sc=False, needs_layout_passes=False
    ),
)
def _embedding_bag_sc(inp_hbm, offs_hbm, psw_hbm, table_hbm, out_hbm,
                      inp_v, offs_v, psw_v, rows_v, cs_v, out_v,
                      gsems, osems):
    _body(inp_hbm, offs_hbm, psw_hbm, table_hbm, out_hbm,
          inp_v, offs_v, psw_v, rows_v, cs_v, out_v,
          gsems, osems)


def kernel(input, offsets, per_sample_weights, weight):
    inp_pad = jnp.pad(input.reshape(B, 2, HALF), ((0, 0), (0, 0), (0, HPAD - HALF)))
    psw_pad = jnp.pad(per_sample_weights, ((0, 0), (0, LANES)))
    score = _embedding_bag_sc(inp_pad, offsets, psw_pad, weight)
    return score, jnp.float32(0.0)


# speedup vs baseline: 1.4016x; 1.4012x over previous
import functools

import jax
import jax.numpy as jnp
from jax import lax
from jax.experimental import pallas as pl
from jax.experimental.pallas import tpu as pltpu, tpu_sc as plsc

B = 4096
N = 200
M = 26
DIM = 64
NC = 2
NS = 16
NW = NC * NS
RPW = B // NW
SPLIT = 104
LANES = 16
NCH = DIM // LANES
NB = 4

def _body(inp_hbm, offs_hbm, psw_hbm, table_hbm, out_hbm,
          inp_v, offs_v, psw_v, rows_v, cs_v, out_v,
          gsems, osems):
    wid = lax.axis_index("s") * NC + lax.axis_index("c")
    base = wid * RPW

    pltpu.sync_copy(inp_hbm.at[pl.ds(base, RPW)], inp_v)
    pltpu.sync_copy(offs_hbm.at[pl.ds(base, RPW)], offs_v)
    pltpu.sync_copy(psw_hbm.at[pl.ds(base, RPW)], psw_v)

    zero = jnp.zeros((LANES,), jnp.float32)
    lanes = lax.iota(jnp.int32, LANES)
    _bcast_dn = lax.GatherDimensionNumbers(
        offset_dims=(), collapsed_slice_dims=(0,), start_index_map=(0,)
    )

    def splat(x):
        return jnp.full((LANES,), x, jnp.int32)

    def bcast_lane(v, l):
        idx = jnp.full((LANES, 1), l, jnp.int32)
        return lax.gather(v, idx, _bcast_dn, (1,),
                          mode=lax.GatherScatterMode.PROMISE_IN_BOUNDS)

    for c in range(NCH):
        cs_v[0, pl.ds(LANES * c, LANES)] = zero

    _spans = ((0, SPLIT), (SPLIT, N - SPLIT))

    def start_gather(r, buf):
        for lo, ln in _spans:
            pltpu.async_copy(
                table_hbm.at[inp_v.at[r, pl.ds(lo, ln)]],
                rows_v.at[buf, pl.ds(lo, ln)],
                gsems[buf],
            )

    def wait_gather(buf):
        for lo, ln in _spans:
            pltpu.make_async_copy(
                table_hbm.at[pl.ds(0, ln)],
                rows_v.at[buf, pl.ds(lo, ln)],
                gsems[buf],
            ).wait()

    for b in range(NB - 1):
        start_gather(b, b)

    def g_body(g, _):
        for p in range(NB):
            r = NB * g + p
            phase = p % 2
            wait_gather(p)

            @pl.when(r + NB - 1 < RPW)
            def _():
                start_gather(r + NB - 1, (p + NB - 1) % NB)

            accs = [zero] * NCH
            for chunk in range((N + LANES - 1) // LANES):
                jbase = chunk * LANES
                cnt = min(LANES, N - jbase)
                wv16 = psw_v[r, pl.ds(jbase, LANES)]
                for l in range(cnt):
                    j = jbase + l
                    w = bcast_lane(wv16, l)
                    for c in range(NCH):
                        x = rows_v[p, j, pl.ds(LANES * c, LANES)]
                        a = accs[c] + x * w
                        cs_v[j + 1, pl.ds(LANES * c, LANES)] = a
                        accs[c] = a

            @pl.when(r >= 2)
            def _():
                pltpu.make_async_copy(
                    out_v.at[phase], out_hbm.at[base], osems[phase]
                ).wait()
            prev = [zero] * NCH
            for m in range(M):
                offm = plsc.load_gather(offs_v, [splat(r), splat(m)]) + 1
                for c in range(NCH):
                    cur = plsc.load_gather(cs_v, [offm, lanes + LANES * c])
                    out_v[phase, m, pl.ds(LANES * c, LANES)] = cur - prev[c]
                    prev[c] = cur
            pltpu.async_copy(out_v.at[phase], out_hbm.at[base + r], osems[phase])
        return 0

    lax.fori_loop(0, RPW // NB, g_body, 0)

    for phase in range(2):
        pltpu.make_async_copy(out_v.at[phase], out_hbm.at[base], osems[phase]).wait()


@functools.partial(
    pl.kernel,
    out_type=jax.ShapeDtypeStruct((B, M, DIM), jnp.float32),
    mesh=plsc.VectorSubcoreMesh(
        core_axis_name="c", subcore_axis_name="s", num_cores=NC, num_subcores=NS
    ),
    scratch_types=[
        pltpu.VMEM((RPW, N), jnp.int32),
        pltpu.VMEM((RPW, M), jnp.int32),
        pltpu.VMEM((RPW, N + LANES), jnp.float32),
        pltpu.VMEM((NB, N, DIM), jnp.float32),
        pltpu.VMEM((N + 4, DIM), jnp.float32),
        pltpu.VMEM((2, M, DIM), jnp.float32),
        [pltpu.SemaphoreType.DMA] * NB,
        [pltpu.SemaphoreType.DMA] * 2,
    ],
    compiler_params=pltpu.CompilerParams(
        use_tc_tiling_on_sc=False, needs_layout_passes=False
    ),
)
def _embedding_bag_sc(inp_hbm, offs_hbm, psw_hbm, table_hbm, out_hbm,
                      inp_v, offs_v, psw_v, rows_v, cs_v, out_v,
                      gsems, osems):
    _body(inp_hbm, offs_hbm, psw_hbm, table_hbm, out_hbm,
          inp_v, offs_v, psw_v, rows_v, cs_v, out_v,
          gsems, osems)


def kernel(input, offsets, per_sample_weights, weight):
    psw_pad = jnp.pad(per_sample_weights, ((0, 0), (0, LANES)))
    score = _embedding_bag_sc(input, offsets, psw_pad, weight)
    return score, jnp.float32(0.0)
